# Initial kernel scaffold; baseline (speedup 1.0000x reference)
import jax
import jax.numpy as jnp
from jax.experimental import pallas as pl


def kernel(x, edge_index, batch, W1, b1, W2, b2, W3, b3, M1, mb1, gamma, beta, M2, mb2):
    raise NotImplementedError("write your pallas kernel here")


# speedup vs baseline: 14.2556x; 14.2556x over previous
import functools

import jax
import jax.numpy as jnp
from jax import lax
from jax.experimental import pallas as pl
from jax.experimental.pallas import tpu as pltpu
from jax.experimental.pallas import tpu_sc as plsc

NN = 10000
NP = 10240
DUMMY = 10176
EE = 320000
NC, NS = 2, 16
NW = NC * NS
CHUNK = 128
CPT = 79
EPT = CPT * CHUNK
EP = NW * EPT
GS = 64
STRIPE = NP // NS

_mesh = plsc.VectorSubcoreMesh(core_axis_name="c", subcore_axis_name="s")
_sc_params = pltpu.CompilerParams(use_tc_tiling_on_sc=False)


def _make_deg():
    @functools.partial(
        pl.kernel,
        out_type=jax.ShapeDtypeStruct((NC, NP, 8), jnp.float32),
        mesh=_mesh,
        scratch_types=[
            pltpu.VMEM((CHUNK,), jnp.int32),
            pltpu.VMEM((CHUNK, 8), jnp.float32),
            pltpu.VMEM_SHARED((NP, 8), jnp.float32),
        ],
        compiler_params=_sc_params,
    )
    def deg_kernel(dst_hbm, ones_hbm, zeros_hbm, out_hbm, didx, ones_v, acc):
        cid = lax.axis_index("c")
        sid = lax.axis_index("s")
        wid = cid * NS + sid
        pltpu.sync_copy(zeros_hbm, acc.at[pl.ds(sid * STRIPE, STRIPE)])
        pltpu.sync_copy(ones_hbm, ones_v)
        plsc.subcore_barrier()
        base = wid * EPT

        def body(i, carry):
            off = base + i * CHUNK
            pltpu.sync_copy(dst_hbm.at[pl.ds(off, CHUNK)], didx)
            pltpu.sync_copy(ones_v, acc.at[didx], add=True)
            return carry

        lax.fori_loop(0, CPT, body, 0)
        plsc.subcore_barrier()
        pltpu.sync_copy(acc.at[pl.ds(sid * STRIPE, STRIPE)],
                        out_hbm.at[cid, pl.ds(sid * STRIPE, STRIPE)])

    return deg_kernel


def _make_agg(width):
    @functools.partial(
        pl.kernel,
        out_type=jax.ShapeDtypeStruct((NC, NP, width), jnp.float32),
        mesh=_mesh,
        scratch_types=[
            pltpu.VMEM((CHUNK,), jnp.int32),
            pltpu.VMEM((CHUNK,), jnp.int32),
            pltpu.VMEM((CHUNK, width), jnp.float32),
            pltpu.VMEM_SHARED((NP, width), jnp.float32),
            pltpu.SemaphoreType.DMA,
        ],
        compiler_params=_sc_params,
    )
    def agg_kernel(y_hbm, src_hbm, dst_hbm, zeros_hbm, out_hbm,
                   sidx, didx, rows, acc, sem):
        cid = lax.axis_index("c")
        sid = lax.axis_index("s")
        wid = cid * NS + sid
        pltpu.sync_copy(zeros_hbm, acc.at[pl.ds(sid * STRIPE, STRIPE)])
        plsc.subcore_barrier()
        base = wid * EPT

        def body(i, carry):
            off = base + i * CHUNK
            pltpu.sync_copy(src_hbm.at[pl.ds(off, CHUNK)], sidx)
            pltpu.async_copy(y_hbm.at[sidx], rows, sem).wait()
            pltpu.sync_copy(dst_hbm.at[pl.ds(off, CHUNK)], didx)
            pltpu.sync_copy(rows, acc.at[didx], add=True)
            return carry

        lax.fori_loop(0, CPT, body, 0)
        plsc.subcore_barrier()
        pltpu.sync_copy(acc.at[pl.ds(sid * STRIPE, STRIPE)],
                        out_hbm.at[cid, pl.ds(sid * STRIPE, STRIPE)])

    return agg_kernel


def _tc_pre(deg_ref, x_ref, w_ref, y_ref, dinv_ref):
    deg = deg_ref[0, :, 0:1] + deg_ref[1, :, 0:1] + 1.0
    dinv = 1.0 / jnp.sqrt(deg)
    dinv_ref[...] = dinv
    xw = jnp.dot(x_ref[...], w_ref[...], preferred_element_type=jnp.float32)
    y_ref[...] = xw * dinv


def _tc_mid(s_ref, y_ref, dinv_ref, b_ref, w_ref, out_ref):
    dinv = dinv_ref[...]
    h = (s_ref[0] + s_ref[1] + y_ref[...]) * dinv + b_ref[...]
    h = jnp.maximum(h, 0.0)
    hw = jnp.dot(h, w_ref[...], preferred_element_type=jnp.float32)
    out_ref[...] = hw * dinv


def _tc_fin(s_ref, y_ref, dinv_ref, b_ref, batch_ref, m1_ref, mb1_ref,
            gamma_ref, beta_ref, m2_ref, mb2_ref, out_ref):
    h = (s_ref[0] + s_ref[1] + y_ref[...]) * dinv_ref[...] + b_ref[...]
    seg = lax.broadcasted_iota(jnp.int32, (NP, GS), 1)
    onehot = (batch_ref[...] == seg).astype(jnp.float32)
    dn = (((0,), (0,)), ((), ()))
    sums = lax.dot_general(onehot, h, dn,
                           preferred_element_type=jnp.float32,
                           precision=lax.Precision.HIGHEST)
    cnt = lax.dot_general(onehot, jnp.ones((NP, 1), jnp.float32), dn,
                          preferred_element_type=jnp.float32,
                          precision=lax.Precision.HIGHEST)
    g = sums / jnp.maximum(cnt, 1.0)
    z = jnp.dot(g, m1_ref[...], preferred_element_type=jnp.float32) + mb1_ref[...]
    mu = jnp.mean(z, axis=0, keepdims=True)
    var = jnp.mean((z - mu) * (z - mu), axis=0, keepdims=True)
    z = (z - mu) / jnp.sqrt(var + 1e-5) * gamma_ref[...] + beta_ref[...]
    z = jnp.maximum(z, 0.0)
    out_ref[...] = (jnp.dot(z, m2_ref[...], preferred_element_type=jnp.float32)
                    + mb2_ref[...])


def kernel(x, edge_index, batch, W1, b1, W2, b2, W3, b3,
           M1, mb1, gamma, beta, M2, mb2):
    f32 = jnp.float32
    src = edge_index[0]
    dst = edge_index[1]
    padv = jnp.full((EP - EE,), DUMMY, jnp.int32)
    srcp = jnp.concatenate([src, padv])
    dstp = jnp.concatenate([dst, padv])
    xp = jnp.pad(x, ((0, NP - NN), (0, 0)))
    batchp = jnp.pad(batch, (0, NP - NN), constant_values=GS)[:, None]

    ones_c = jnp.ones((CHUNK, 8), f32)
    zeros1 = jnp.zeros((STRIPE, 8), f32)
    zeros_h = jnp.zeros((STRIPE, 64), f32)
    zeros_o = jnp.zeros((STRIPE, 16), f32)

    deg2 = _make_deg()(dstp, ones_c, zeros1)

    H, O = W1.shape[1], W3.shape[1]
    y1, dinv = pl.pallas_call(
        _tc_pre,
        out_shape=(jax.ShapeDtypeStruct((NP, H), f32),
                   jax.ShapeDtypeStruct((NP, 1), f32)),
    )(deg2, xp, W1)

    agg_h = _make_agg(H)
    s1 = agg_h(y1, srcp, dstp, zeros_h)
    y2 = pl.pallas_call(
        _tc_mid, out_shape=jax.ShapeDtypeStruct((NP, H), f32),
    )(s1, y1, dinv, b1[None, :], W2)

    s2 = agg_h(y2, srcp, dstp, zeros_h)
    y3 = pl.pallas_call(
        _tc_mid, out_shape=jax.ShapeDtypeStruct((NP, O), f32),
    )(s2, y2, dinv, b2[None, :], W3)

    s3 = _make_agg(O)(y3, srcp, dstp, zeros_o)
    out = pl.pallas_call(
        _tc_fin, out_shape=jax.ShapeDtypeStruct((GS, 2), f32),
    )(s3, y3, dinv, b3[None, :], batchp, M1, mb1[None, :],
      gamma[None, :], beta[None, :], M2, mb2[None, :])
    return out


# speedup vs baseline: 19.8347x; 1.3914x over previous
import functools

import jax
import jax.numpy as jnp
from jax import lax
from jax.experimental import pallas as pl
from jax.experimental.pallas import tpu as pltpu
from jax.experimental.pallas import tpu_sc as plsc

NN = 10000
NP = 10240
DUMMY = 10176
EE = 320000
NC, NS = 2, 16
NW = NC * NS
CHUNK = 128
CPT = 80
EPT = CPT * CHUNK
EP = NW * EPT
NBUF = 4
GS = 64
STRIPE = NP // NS

_mesh = plsc.VectorSubcoreMesh(core_axis_name="c", subcore_axis_name="s")
_sc_params = pltpu.CompilerParams(use_tc_tiling_on_sc=False)


def _make_deg():
    @functools.partial(
        pl.kernel,
        out_type=jax.ShapeDtypeStruct((NC, NP, 8), jnp.float32),
        mesh=_mesh,
        scratch_types=[
            pltpu.VMEM((CPT, CHUNK), jnp.int32),
            pltpu.VMEM((CHUNK, 8), jnp.float32),
            pltpu.VMEM_SHARED((NP, 8), jnp.float32),
        ],
        compiler_params=_sc_params,
    )
    def deg_kernel(dst_hbm, ones_hbm, zeros_hbm, out_hbm, didx, ones_v, acc):
        cid = lax.axis_index("c")
        sid = lax.axis_index("s")
        wid = cid * NS + sid
        pltpu.sync_copy(zeros_hbm, acc.at[pl.ds(sid * STRIPE, STRIPE)])
        pltpu.sync_copy(ones_hbm, ones_v)
        pltpu.sync_copy(dst_hbm.at[pl.ds(wid * CPT, CPT)], didx)
        plsc.subcore_barrier()

        def body(i, carry):
            pltpu.sync_copy(ones_v, acc.at[didx.at[i]], add=True)
            return carry

        lax.fori_loop(0, CPT, body, 0)
        plsc.subcore_barrier()
        pltpu.sync_copy(acc.at[pl.ds(sid * STRIPE, STRIPE)],
                        out_hbm.at[cid, pl.ds(sid * STRIPE, STRIPE)])

    return deg_kernel


def _make_agg(width):
    @functools.partial(
        pl.kernel,
        out_type=jax.ShapeDtypeStruct((NC, NP, width), jnp.float32),
        mesh=_mesh,
        scratch_types=[
            pltpu.VMEM((CPT, CHUNK), jnp.int32),
            pltpu.VMEM((CPT, CHUNK), jnp.int32),
            [pltpu.VMEM((CHUNK, width), jnp.float32) for _ in range(NBUF)],
            pltpu.VMEM_SHARED((NP, width), jnp.float32),
            [pltpu.SemaphoreType.DMA for _ in range(NBUF)],
        ],
        compiler_params=_sc_params,
    )
    def agg_kernel(y_hbm, src_hbm, dst_hbm, zeros_hbm, out_hbm,
                   sidx, didx, rows, acc, sems):
        cid = lax.axis_index("c")
        sid = lax.axis_index("s")
        wid = cid * NS + sid
        pltpu.sync_copy(zeros_hbm, acc.at[pl.ds(sid * STRIPE, STRIPE)])
        pltpu.sync_copy(src_hbm.at[pl.ds(wid * CPT, CPT)], sidx)
        pltpu.sync_copy(dst_hbm.at[pl.ds(wid * CPT, CPT)], didx)
        plsc.subcore_barrier()
        for k in range(NBUF):
            pltpu.async_copy(y_hbm.at[sidx.at[k]], rows[k], sems[k])

        def round_body(r, carry):
            c0 = r * NBUF
            for k in range(NBUF):
                pltpu.make_async_copy(y_hbm.at[sidx.at[c0 + k]],
                                      rows[k], sems[k]).wait()
                pltpu.sync_copy(rows[k], acc.at[didx.at[c0 + k]], add=True)

                @pl.when(r < CPT // NBUF - 1)
                def _():
                    pltpu.async_copy(y_hbm.at[sidx.at[c0 + NBUF + k]],
                                     rows[k], sems[k])
            return carry

        lax.fori_loop(0, CPT // NBUF, round_body, 0)
        plsc.subcore_barrier()
        pltpu.sync_copy(acc.at[pl.ds(sid * STRIPE, STRIPE)],
                        out_hbm.at[cid, pl.ds(sid * STRIPE, STRIPE)])

    return agg_kernel


def _tc_pre(deg_ref, x_ref, w_ref, y_ref, dinv_ref):
    deg = deg_ref[0, :, 0:1] + deg_ref[1, :, 0:1] + 1.0
    dinv = 1.0 / jnp.sqrt(deg)
    dinv_ref[...] = dinv
    xw = jnp.dot(x_ref[...], w_ref[...], preferred_element_type=jnp.float32)
    y_ref[...] = xw * dinv


def _tc_mid(s_ref, y_ref, dinv_ref, b_ref, w_ref, out_ref):
    dinv = dinv_ref[...]
    h = (s_ref[0] + s_ref[1] + y_ref[...]) * dinv + b_ref[...]
    h = jnp.maximum(h, 0.0)
    hw = jnp.dot(h, w_ref[...], preferred_element_type=jnp.float32)
    out_ref[...] = hw * dinv


def _tc_fin(s_ref, y_ref, dinv_ref, b_ref, batch_ref, m1_ref, mb1_ref,
            gamma_ref, beta_ref, m2_ref, mb2_ref, out_ref):
    h = (s_ref[0] + s_ref[1] + y_ref[...]) * dinv_ref[...] + b_ref[...]
    seg = lax.broadcasted_iota(jnp.int32, (NP, GS), 1)
    onehot = (batch_ref[...] == seg).astype(jnp.float32)
    dn = (((0,), (0,)), ((), ()))
    sums = lax.dot_general(onehot, h, dn,
                           preferred_element_type=jnp.float32,
                           precision=lax.Precision.HIGHEST)
    cnt = lax.dot_general(onehot, jnp.ones((NP, 1), jnp.float32), dn,
                          preferred_element_type=jnp.float32,
                          precision=lax.Precision.HIGHEST)
    g = sums / jnp.maximum(cnt, 1.0)
    z = jnp.dot(g, m1_ref[...], preferred_element_type=jnp.float32) + mb1_ref[...]
    mu = jnp.mean(z, axis=0, keepdims=True)
    var = jnp.mean((z - mu) * (z - mu), axis=0, keepdims=True)
    z = (z - mu) / jnp.sqrt(var + 1e-5) * gamma_ref[...] + beta_ref[...]
    z = jnp.maximum(z, 0.0)
    out_ref[...] = (jnp.dot(z, m2_ref[...], preferred_element_type=jnp.float32)
                    + mb2_ref[...])


def kernel(x, edge_index, batch, W1, b1, W2, b2, W3, b3,
           M1, mb1, gamma, beta, M2, mb2):
    f32 = jnp.float32
    src = edge_index[0]
    dst = edge_index[1]
    padv = jnp.full((EP - EE,), DUMMY, jnp.int32)
    srcp = jnp.concatenate([src, padv]).reshape(NW * CPT, CHUNK)
    dstp = jnp.concatenate([dst, padv]).reshape(NW * CPT, CHUNK)
    xp = jnp.pad(x, ((0, NP - NN), (0, 0)))
    batchp = jnp.pad(batch, (0, NP - NN), constant_values=GS)[:, None]

    ones_c = jnp.ones((CHUNK, 8), f32)
    zeros1 = jnp.zeros((STRIPE, 8), f32)
    zeros_h = jnp.zeros((STRIPE, 64), f32)
    zeros_o = jnp.zeros((STRIPE, 16), f32)

    deg2 = _make_deg()(dstp, ones_c, zeros1)

    H, O = W1.shape[1], W3.shape[1]
    y1, dinv = pl.pallas_call(
        _tc_pre,
        out_shape=(jax.ShapeDtypeStruct((NP, H), f32),
                   jax.ShapeDtypeStruct((NP, 1), f32)),
    )(deg2, xp, W1)

    agg_h = _make_agg(H)
    s1 = agg_h(y1, srcp, dstp, zeros_h)
    y2 = pl.pallas_call(
        _tc_mid, out_shape=jax.ShapeDtypeStruct((NP, H), f32),
    )(s1, y1, dinv, b1[None, :], W2)

    s2 = agg_h(y2, srcp, dstp, zeros_h)
    y3 = pl.pallas_call(
        _tc_mid, out_shape=jax.ShapeDtypeStruct((NP, O), f32),
    )(s2, y2, dinv, b2[None, :], W3)

    s3 = _make_agg(O)(y3, srcp, dstp, zeros_o)
    out = pl.pallas_call(
        _tc_fin, out_shape=jax.ShapeDtypeStruct((GS, 2), f32),
    )(s3, y3, dinv, b3[None, :], batchp, M1, mb1[None, :],
      gamma[None, :], beta[None, :], M2, mb2[None, :])
    return out
